# baseline (speedup 1.0000x reference)
import functools

import jax
import jax.numpy as jnp
from jax import lax
from jax.experimental import pallas as pl
from jax.experimental.pallas import tpu as pltpu
from jax.experimental.pallas import tpu_sc as plsc

N = 10000
E = 320000
NW = 32
NPW = 313
NPW_PAD = 512
MULC = 107203
SHIFT = 25
EPAD = 327680
B_POS = 512
B_E = 2000
B_MSG = 2048
B_N = 2000
CP = 400
CG = 512
CM = 256
USERS = 10
INT_MIN = -2147483647

def _mesh():
    return plsc.VectorSubcoreMesh(core_axis_name="c", subcore_axis_name="s")


def _wid():
    return lax.axis_index("s") * 2 + lax.axis_index("c")


def _sget(ref, i):
    base = (i // 16) * 16
    win = ref[pl.ds(base, 16)]
    m = lax.iota(jnp.int32, 16) == (i - base)
    return jnp.max(jnp.where(m, win, INT_MIN))


def _hist_body(dst_ref, hist_ref, acc):
    i = pl.program_id(0)

    @pl.when(i == 0)
    def _():
        acc[...] = jnp.zeros_like(acc)

    d = dst_ref[0]
    bucket = (d * MULC) >> SHIFT
    oh = jnp.where(bucket == lax.broadcasted_iota(jnp.int32, (1, NW), 1),
                   1.0, 0.0)
    acc[...] += jnp.sum(oh, axis=0, keepdims=True)

    @pl.when(i == pl.num_programs(0) - 1)
    def _():
        hist_ref[...] = acc[...]


def _k_hist(dst3):
    return pl.pallas_call(
        _hist_body,
        grid=(E // B_POS,),
        in_specs=[pl.BlockSpec((1, B_POS, 1), lambda i: (i, 0, 0))],
        out_specs=pl.BlockSpec((1, NW), lambda i: (0, 0)),
        out_shape=jax.ShapeDtypeStruct((1, NW), jnp.float32),
        scratch_shapes=[pltpu.VMEM((1, NW), jnp.float32)],
    )(dst3)


def _pos_body(dst_ref, hist_ref, lt_ref, lt32_ref,
              pos_ref, dstloc_ref, off_ref, cnt_ref, run):
    i = pl.program_id(0)
    hist_i = hist_ref[...].astype(jnp.int32)
    cnt_al = (hist_i + 15) & jnp.int32(-16)
    cnt_al_f = cnt_al.astype(jnp.float32)
    off_ex_f = jnp.dot(cnt_al_f, lt32_ref[...],
                       preferred_element_type=jnp.float32)

    @pl.when(i == 0)
    def _():
        run[...] = off_ex_f
        z = jnp.zeros((1, 48 - NW - 1), jnp.float32)
        total = jnp.sum(cnt_al_f, axis=1, keepdims=True)
        off_ref[...] = jnp.concatenate([off_ex_f, total, z], axis=1).astype(jnp.int32)
        zc = jnp.zeros((1, 48 - NW), jnp.float32)
        cnt_ref[...] = jnp.concatenate([hist_ref[...], zc], axis=1).astype(jnp.int32)

    d = dst_ref[0]
    bucket = (d * MULC) >> SHIFT
    oh = jnp.where(bucket == lax.broadcasted_iota(jnp.int32, (1, NW), 1),
                   1.0, 0.0)
    rank = jnp.dot(lt_ref[...], oh, preferred_element_type=jnp.float32)
    posv = jnp.sum((rank + run[...]) * oh, axis=1, keepdims=True)
    pos_ref[0] = posv.astype(jnp.int32)
    dstloc_ref[0] = d - bucket * jnp.int32(NPW)
    run[...] += jnp.sum(oh, axis=0, keepdims=True)


def _k_pos(dst3, hist, lt, lt32):
    nb = E // B_POS
    return pl.pallas_call(
        _pos_body,
        grid=(nb,),
        in_specs=[
            pl.BlockSpec((1, B_POS, 1), lambda i: (i, 0, 0)),
            pl.BlockSpec((1, NW), lambda i: (0, 0)),
            pl.BlockSpec((B_POS, B_POS), lambda i: (0, 0)),
            pl.BlockSpec((NW, NW), lambda i: (0, 0)),
        ],
        out_specs=[
            pl.BlockSpec((1, B_POS, 1), lambda i: (i, 0, 0)),
            pl.BlockSpec((1, B_POS, 1), lambda i: (i, 0, 0)),
            pl.BlockSpec((1, 48), lambda i: (0, 0)),
            pl.BlockSpec((1, 48), lambda i: (0, 0)),
        ],
        out_shape=[
            jax.ShapeDtypeStruct((nb, B_POS, 1), jnp.int32),
            jax.ShapeDtypeStruct((nb, B_POS, 1), jnp.int32),
            jax.ShapeDtypeStruct((1, 48), jnp.int32),
            jax.ShapeDtypeStruct((1, 48), jnp.int32),
        ],
        scratch_shapes=[pltpu.VMEM((1, NW), jnp.float32)],
    )(dst3, hist, lt, lt32)


def _pad_body(ea_ref, out_ref):
    z = jnp.zeros((B_E, 128 - 65), jnp.float32)
    out_ref[...] = jnp.concatenate([ea_ref[...], z], axis=1)


def _k_pad(ea):
    return pl.pallas_call(
        _pad_body,
        grid=(E // B_E,),
        in_specs=[pl.BlockSpec((B_E, 65), lambda i: (i, 0))],
        out_specs=pl.BlockSpec((B_E, 128), lambda i: (i, 0)),
        out_shape=jax.ShapeDtypeStruct((E, 128), jnp.float32),
    )(ea)


def _e_body(ea_ref, a2_ref, e_ref):
    e_ref[...] = jnp.dot(ea_ref[:, 0:65], a2_ref[...],
                         preferred_element_type=jnp.float32)


def _k_e(ea_b, a2):
    return pl.pallas_call(
        _e_body,
        grid=(EPAD // B_MSG,),
        in_specs=[
            pl.BlockSpec((B_MSG, 128), lambda i: (i, 0)),
            pl.BlockSpec((65, 64), lambda i: (0, 0)),
        ],
        out_specs=pl.BlockSpec((B_MSG, 64), lambda i: (i, 0)),
        out_shape=jax.ShapeDtypeStruct((EPAD, 64), jnp.float32),
    )(ea_b, a2)


def _perm_body(src_hbm, dstloc_hbm, pos_hbm,
               invp_hbm, srcb_hbm, dstlocb_hbm,
               posb, sbuf, dbuf, ibuf, sem):
    base = _wid() * (E // NW)
    iota = lax.iota(jnp.int32, 16)
    for c in range(E // NW // CP):
        st = base + c * CP
        pltpu.sync_copy(pos_hbm.at[pl.ds(st, CP)], posb)
        pltpu.sync_copy(src_hbm.at[pl.ds(st, CP)], sbuf)
        pltpu.sync_copy(dstloc_hbm.at[pl.ds(st, CP)], dbuf)
        for g in range(CP // 16):
            ibuf[pl.ds(g * 16, 16)] = iota + jnp.int32(st + g * 16)
        pltpu.async_copy(ibuf, invp_hbm.at[posb], sem).wait()
        pltpu.async_copy(sbuf, srcb_hbm.at[posb], sem).wait()
        pltpu.async_copy(dbuf, dstlocb_hbm.at[posb], sem).wait()


def _k_perm(src, dstloc, pos):
    f = pl.kernel(
        _perm_body, mesh=_mesh(),
        compiler_params=pltpu.CompilerParams(needs_layout_passes=False),
        out_type=(
            jax.ShapeDtypeStruct((EPAD,), jnp.int32),
            jax.ShapeDtypeStruct((EPAD,), jnp.int32),
            jax.ShapeDtypeStruct((EPAD,), jnp.int32),
        ),
        scratch_types=[
            pltpu.VMEM((CP,), jnp.int32),
            pltpu.VMEM((CP,), jnp.int32),
            pltpu.VMEM((CP,), jnp.int32),
            pltpu.VMEM((CP,), jnp.int32),
            pltpu.SemaphoreType.DMA,
        ],
    )
    return f(src, dstloc, pos)


def _rowgather_body(nrows, table_hbm, idx_hbm, out_hbm, idxb, idxc, rows, sem):
    base = _wid() * (EPAD // NW)
    for c in range(EPAD // NW // CG):
        st = base + c * CG
        pltpu.sync_copy(idx_hbm.at[pl.ds(st, CG)], idxb)
        for g in range(CG // 16):
            v = idxb[pl.ds(g * 16, 16)]
            v = jnp.minimum(jnp.maximum(v, 0), jnp.int32(nrows - 1))
            idxc[pl.ds(g * 16, 16)] = v
        pltpu.async_copy(table_hbm.at[idxc], rows, sem).wait()
        pltpu.sync_copy(rows, out_hbm.at[pl.ds(st, CG)])


def _k_rowgather(table, idx):
    f = pl.kernel(
        functools.partial(_rowgather_body, table.shape[0]), mesh=_mesh(),
        compiler_params=pltpu.CompilerParams(needs_layout_passes=False),
        out_type=jax.ShapeDtypeStruct((EPAD, 128), jnp.float32),
        scratch_types=[
            pltpu.VMEM((CG,), jnp.int32),
            pltpu.VMEM((CG,), jnp.int32),
            pltpu.VMEM((CG, 128), jnp.float32),
            pltpu.SemaphoreType.DMA,
        ],
    )
    return f(table, idx)


def _msg_body(xg_ref, eb_ref, a1_ref, b1a_ref, w1b_ref, b1b_ref, msg_ref):
    h = jnp.dot(xg_ref[...], a1_ref[...], preferred_element_type=jnp.float32)
    h = jnp.maximum(h + eb_ref[...] + b1a_ref[...], 0.0)
    m = jnp.dot(h, w1b_ref[...], preferred_element_type=jnp.float32) + b1b_ref[...]
    msg_ref[...] = jnp.maximum(m, 0.0)


def _k_msg(xg, eb, a1, b1a_row, w1b, b1b_row):
    return pl.pallas_call(
        _msg_body,
        grid=(EPAD // B_MSG,),
        in_specs=[
            pl.BlockSpec((B_MSG, 128), lambda i: (i, 0)),
            pl.BlockSpec((B_MSG, 64), lambda i: (i, 0)),
            pl.BlockSpec((128, 64), lambda i: (0, 0)),
            pl.BlockSpec((1, 64), lambda i: (0, 0)),
            pl.BlockSpec((64, 64), lambda i: (0, 0)),
            pl.BlockSpec((1, 64), lambda i: (0, 0)),
        ],
        out_specs=pl.BlockSpec((B_MSG, 64), lambda i: (i, 0)),
        out_shape=jax.ShapeDtypeStruct((EPAD, 64), jnp.float32),
    )(xg, eb, a1, b1a_row, w1b, b1b_row)


def _max_body(msg_hbm, dstlocb_hbm, off_hbm, cnt_hbm, aggr_hbm,
              aggr_v, msgb, dlb, offb, cntb, tmp_v, sem):
    w = _wid()
    pltpu.sync_copy(off_hbm, offb)
    pltpu.sync_copy(cnt_hbm, cntb)
    my_off = _sget(offb, w)
    my_cnt = _sget(cntb, w)

    def _zero(i, _):
        aggr_v[pl.ds(i * 16, 16)] = jnp.zeros((16,), jnp.float32)
        return 0

    lax.fori_loop(0, NPW_PAD * 64 // 16, _zero, 0)

    iota = lax.iota(jnp.int32, 16)
    nchunks = (my_cnt + CM - 1) // CM

    def _chunk(c, _):
        st = pl.multiple_of(my_off + c * CM, 16)
        pltpu.sync_copy(msg_hbm.at[pl.ds(st, CM)], msgb)
        pltpu.sync_copy(dstlocb_hbm.at[pl.ds(st, CM)], dlb)
        nvalid = jnp.minimum(my_cnt - c * CM, CM)
        for g in range(CM // 16):
            d = dlb[pl.ds(g * 16, 16)] & jnp.int32(NPW_PAD - 1)
            valid = iota < (nvalid - g * 16)
            rows = iota + jnp.int32(g * 16)

            def _cond(carry):
                done = carry
                return jnp.sum(jnp.where(valid & (done == 0), 1, 0)) > 0

            def _pass(carry):
                done = carry
                rem = valid & (done == 0)
                plsc.store_scatter(tmp_v, [d], iota, mask=rem)
                rb = plsc.load_gather(tmp_v, [d])
                win = rem & (rb == iota)

                def _feat(j8, _):
                    for jj in range(8):
                        j = j8 * 8 + jj
                        col = jnp.full((16,), j, jnp.int32)
                        mv = plsc.load_gather(msgb, [rows, col])
                        cur = plsc.load_gather(aggr_v, [d * 64 + j])
                        plsc.store_scatter(aggr_v, [d * 64 + j],
                                           jnp.maximum(cur, mv), mask=win)
                    return 0

                lax.fori_loop(0, 8, _feat, 0)
                return done + jnp.where(win, 1, 0)

            lax.while_loop(_cond, _pass, jnp.zeros((16,), jnp.int32))
        return 0

    lax.fori_loop(0, nchunks, _chunk, 0)
    pltpu.sync_copy(aggr_v.at[pl.ds(0, NPW * 64)],
                    aggr_hbm.at[pl.ds(w * (NPW * 64), NPW * 64)])


def _k_max(msg, dstlocb, off48, cnt48):
    f = pl.kernel(
        _max_body, mesh=_mesh(),
        compiler_params=pltpu.CompilerParams(needs_layout_passes=False),
        out_type=jax.ShapeDtypeStruct((NW * NPW * 64,), jnp.float32),
        scratch_types=[
            pltpu.VMEM((NPW_PAD * 64,), jnp.float32),
            pltpu.VMEM((CM, 64), jnp.float32),
            pltpu.VMEM((CM,), jnp.int32),
            pltpu.VMEM((48,), jnp.int32),
            pltpu.VMEM((48,), jnp.int32),
            pltpu.VMEM((NPW_PAD,), jnp.int32),
            pltpu.SemaphoreType.DMA,
        ],
    )
    return f(msg, dstlocb, off48, cnt48)


def _node_body(x_ref, aggr_ref, b1_ref, b2_ref, b2a_ref, w2b_ref, b2b_ref,
               g_ref, wsel_ref, out_ref):
    x = x_ref[...]
    h2 = jnp.dot(x, b1_ref[...], preferred_element_type=jnp.float32)
    h2 += jnp.dot(aggr_ref[...], b2_ref[...], preferred_element_type=jnp.float32)
    h2 = jnp.maximum(h2 + b2a_ref[...], 0.0)
    ca = jnp.dot(h2, w2b_ref[...], preferred_element_type=jnp.float32) + b2b_ref[...]
    links = ca[:, 0:1]
    addd = ca[:, 1:2]
    comb = ca[:, 2:66]
    sqrow = jnp.sum(comb * comb, axis=1, keepdims=True)
    g = g_ref[...]
    per_grp = lax.dot_general(g, sqrow, (((0,), (0,)), ((), ())),
                              preferred_element_type=jnp.float32)
    sq_g = jnp.dot(g, per_grp, preferred_element_type=jnp.float32)
    scale = lax.rsqrt(sq_g + 1e-06)
    combn = comb * scale
    out = jnp.concatenate([links, addd, combn, x[:, 0:62]], axis=1)
    wsel = wsel_ref[0]
    out = jnp.where(wsel > 0.0, out, x)
    out_ref[...] = out


def _k_node(x, aggr, b1, b2, b2a_row, w2b, b2b_row, g, wsel):
    return pl.pallas_call(
        _node_body,
        grid=(N // B_N,),
        in_specs=[
            pl.BlockSpec((B_N, 128), lambda i: (i, 0)),
            pl.BlockSpec((B_N, 64), lambda i: (i, 0)),
            pl.BlockSpec((128, 32), lambda i: (0, 0)),
            pl.BlockSpec((64, 32), lambda i: (0, 0)),
            pl.BlockSpec((1, 32), lambda i: (0, 0)),
            pl.BlockSpec((32, 66), lambda i: (0, 0)),
            pl.BlockSpec((1, 66), lambda i: (0, 0)),
            pl.BlockSpec((B_N, B_N // USERS), lambda i: (0, 0)),
            pl.BlockSpec(memory_space=pltpu.SMEM),
        ],
        out_specs=pl.BlockSpec((B_N, 128), lambda i: (i, 0)),
        out_shape=jax.ShapeDtypeStruct((N, 128), jnp.float32),
    )(x, aggr, b1, b2, b2a_row, w2b, b2b_row, g, wsel)


def kernel(x, edge_index, edge_attr, weights, W1a, b1a, W1b, b1b,
           W2a, b2a, W2b, b2b):
    src = edge_index[0]
    dst = edge_index[1]
    dst3 = dst.reshape(E // B_POS, B_POS, 1)
    a1 = W1a[:128]
    a2 = W1a[128:]
    b1 = W2a[:128]
    b2 = W2a[128:]
    b1a_row = b1a.reshape(1, 64)
    b1b_row = b1b.reshape(1, 64)
    b2a_row = b2a.reshape(1, 32)
    b2b_row = b2b.reshape(1, 66)
    lt = jnp.tril(jnp.ones((B_POS, B_POS), jnp.float32), k=-1)
    lt32 = jnp.triu(jnp.ones((NW, NW), jnp.float32), k=1)
    grp = jnp.repeat(jnp.eye(B_N // USERS, dtype=jnp.float32), USERS, axis=0)

    hist = _k_hist(dst3)
    pos3, dstloc3, off48, cnt48 = _k_pos(dst3, hist, lt, lt32)
    pos = pos3.reshape(E)
    dstloc = dstloc3.reshape(E)
    off48 = off48.reshape(48)
    cnt48 = cnt48.reshape(48)
    ea128 = _k_pad(edge_attr)
    invp, srcb, dstlocb = _k_perm(src, dstloc, pos)
    ea_b = _k_rowgather(ea128, invp)
    e_b = _k_e(ea_b, a2)

    xl = x
    for layer in range(2):
        xg = _k_rowgather(xl, srcb)
        msg = _k_msg(xg, e_b, a1, b1a_row, W1b, b1b_row)
        aggr_flat = _k_max(msg, dstlocb, off48, cnt48)
        aggr = aggr_flat.reshape(NW * NPW, 64)[:N]
        wsel = weights[layer:layer + 1]
        xl = _k_node(xl, aggr, b1, b2, b2a_row, W2b, b2b_row, grp, wsel)
    return xl


# speedup vs baseline: 1.4846x; 1.4846x over previous
import functools

import jax
import jax.numpy as jnp
from jax import lax
from jax.experimental import pallas as pl
from jax.experimental.pallas import tpu as pltpu
from jax.experimental.pallas import tpu_sc as plsc

N = 10000
E = 320000
NW = 32
NPW = 313
NPW_PAD = 512
MULC = 107203
SHIFT = 25
EPAD = 327680
B_POS = 512
B_E = 2000
B_MSG = 2048
B_N = 2000
CP = 2000
CG = 512
CM = 256
USERS = 10
INT_MIN = -2147483647

def _mesh():
    return plsc.VectorSubcoreMesh(core_axis_name="c", subcore_axis_name="s")


def _wid():
    return lax.axis_index("s") * 2 + lax.axis_index("c")


def _sget(ref, i):
    base = (i // 16) * 16
    win = ref[pl.ds(base, 16)]
    m = lax.iota(jnp.int32, 16) == (i - base)
    return jnp.max(jnp.where(m, win, INT_MIN))


def _hist_body(dst_ref, hist_ref, acc):
    i = pl.program_id(0)

    @pl.when(i == 0)
    def _():
        acc[...] = jnp.zeros_like(acc)

    d = dst_ref[0]
    bucket = (d * MULC) >> SHIFT
    oh = jnp.where(bucket == lax.broadcasted_iota(jnp.int32, (1, NW), 1),
                   1.0, 0.0)
    acc[...] += jnp.sum(oh, axis=0, keepdims=True)

    @pl.when(i == pl.num_programs(0) - 1)
    def _():
        hist_ref[...] = acc[...]


def _k_hist(dst3):
    return pl.pallas_call(
        _hist_body,
        grid=(E // B_POS,),
        in_specs=[pl.BlockSpec((1, B_POS, 1), lambda i: (i, 0, 0))],
        out_specs=pl.BlockSpec((1, NW), lambda i: (0, 0)),
        out_shape=jax.ShapeDtypeStruct((1, NW), jnp.float32),
        scratch_shapes=[pltpu.VMEM((1, NW), jnp.float32)],
    )(dst3)


def _pos_body(dst_ref, src_ref, hist_ref, lt_ref, lt32_ref,
              pos_ref, pack_ref, off_ref, cnt_ref, run):
    i = pl.program_id(0)
    hist_i = hist_ref[...].astype(jnp.int32)
    cnt_al = (hist_i + 15) & jnp.int32(-16)
    cnt_al_f = cnt_al.astype(jnp.float32)
    off_ex_f = jnp.dot(cnt_al_f, lt32_ref[...],
                       preferred_element_type=jnp.float32)

    @pl.when(i == 0)
    def _():
        run[...] = off_ex_f
        z = jnp.zeros((1, 48 - NW - 1), jnp.float32)
        total = jnp.sum(cnt_al_f, axis=1, keepdims=True)
        off_ref[...] = jnp.concatenate([off_ex_f, total, z], axis=1).astype(jnp.int32)
        zc = jnp.zeros((1, 48 - NW), jnp.float32)
        cnt_ref[...] = jnp.concatenate([hist_ref[...], zc], axis=1).astype(jnp.int32)

    d = dst_ref[0]
    bucket = (d * MULC) >> SHIFT
    oh = jnp.where(bucket == lax.broadcasted_iota(jnp.int32, (1, NW), 1),
                   1.0, 0.0)
    rank = jnp.dot(lt_ref[...], oh, preferred_element_type=jnp.float32)
    posv = jnp.sum((rank + run[...]) * oh, axis=1, keepdims=True)
    pos_ref[0] = posv.astype(jnp.int32)
    pack_ref[0] = src_ref[0] * jnp.int32(512) + (d - bucket * jnp.int32(NPW))
    run[...] += jnp.sum(oh, axis=0, keepdims=True)


def _k_pos(dst3, src3, hist, lt, lt32):
    nb = E // B_POS
    return pl.pallas_call(
        _pos_body,
        grid=(nb,),
        in_specs=[
            pl.BlockSpec((1, B_POS, 1), lambda i: (i, 0, 0)),
            pl.BlockSpec((1, B_POS, 1), lambda i: (i, 0, 0)),
            pl.BlockSpec((1, NW), lambda i: (0, 0)),
            pl.BlockSpec((B_POS, B_POS), lambda i: (0, 0)),
            pl.BlockSpec((NW, NW), lambda i: (0, 0)),
        ],
        out_specs=[
            pl.BlockSpec((1, B_POS, 1), lambda i: (i, 0, 0)),
            pl.BlockSpec((1, B_POS, 1), lambda i: (i, 0, 0)),
            pl.BlockSpec((1, 48), lambda i: (0, 0)),
            pl.BlockSpec((1, 48), lambda i: (0, 0)),
        ],
        out_shape=[
            jax.ShapeDtypeStruct((nb, B_POS, 1), jnp.int32),
            jax.ShapeDtypeStruct((nb, B_POS, 1), jnp.int32),
            jax.ShapeDtypeStruct((1, 48), jnp.int32),
            jax.ShapeDtypeStruct((1, 48), jnp.int32),
        ],
        scratch_shapes=[pltpu.VMEM((1, NW), jnp.float32)],
    )(dst3, src3, hist, lt, lt32)


def _pad_body(ea_ref, out_ref):
    z = jnp.zeros((B_E, 128 - 65), jnp.float32)
    out_ref[...] = jnp.concatenate([ea_ref[...], z], axis=1)


def _k_pad(ea):
    return pl.pallas_call(
        _pad_body,
        grid=(E // B_E,),
        in_specs=[pl.BlockSpec((B_E, 65), lambda i: (i, 0))],
        out_specs=pl.BlockSpec((B_E, 128), lambda i: (i, 0)),
        out_shape=jax.ShapeDtypeStruct((E, 128), jnp.float32),
    )(ea)


def _e_body(ea_ref, a2_ref, e_ref):
    e_ref[...] = jnp.dot(ea_ref[:, 0:65], a2_ref[...],
                         preferred_element_type=jnp.float32)


def _k_e(ea_b, a2):
    return pl.pallas_call(
        _e_body,
        grid=(EPAD // B_MSG,),
        in_specs=[
            pl.BlockSpec((B_MSG, 128), lambda i: (i, 0)),
            pl.BlockSpec((65, 64), lambda i: (0, 0)),
        ],
        out_specs=pl.BlockSpec((B_MSG, 64), lambda i: (i, 0)),
        out_shape=jax.ShapeDtypeStruct((EPAD, 64), jnp.float32),
    )(ea_b, a2)


def _perm_body(pack_hbm, pos_hbm, packb_hbm, invp_hbm, posb, pbuf, ibuf, sem):
    base = _wid() * (E // NW)
    iota = lax.iota(jnp.int32, 16)
    for c in range(E // NW // CP):
        st = base + c * CP
        pltpu.sync_copy(pos_hbm.at[pl.ds(st, CP)], posb)
        pltpu.sync_copy(pack_hbm.at[pl.ds(st, CP)], pbuf)
        for g in range(CP // 16):
            ibuf[pl.ds(g * 16, 16)] = iota + jnp.int32(st + g * 16)
        h1 = pltpu.async_copy(pbuf, packb_hbm.at[posb], sem)
        h2 = pltpu.async_copy(ibuf, invp_hbm.at[posb], sem)
        h1.wait()
        h2.wait()


def _k_perm(pack2, pos):
    f = pl.kernel(
        _perm_body, mesh=_mesh(),
        compiler_params=pltpu.CompilerParams(needs_layout_passes=False),
        out_type=(
            jax.ShapeDtypeStruct((EPAD,), jnp.int32),
            jax.ShapeDtypeStruct((EPAD,), jnp.int32),
        ),
        scratch_types=[
            pltpu.VMEM((CP,), jnp.int32),
            pltpu.VMEM((CP,), jnp.int32),
            pltpu.VMEM((CP,), jnp.int32),
            pltpu.SemaphoreType.DMA,
        ],
    )
    return f(pack2, pos)


def _rowgather_body(nrows, shift, table_hbm, idx_hbm, out_hbm, idxb, idxc, rows, sem):
    base = _wid() * (EPAD // NW)
    for c in range(EPAD // NW // CG):
        st = base + c * CG
        pltpu.sync_copy(idx_hbm.at[pl.ds(st, CG)], idxb)
        for g in range(CG // 16):
            v = idxb[pl.ds(g * 16, 16)] >> shift
            v = jnp.minimum(jnp.maximum(v, 0), jnp.int32(nrows - 1))
            idxc[pl.ds(g * 16, 16)] = v
        pltpu.async_copy(table_hbm.at[idxc], rows, sem).wait()
        pltpu.sync_copy(rows, out_hbm.at[pl.ds(st, CG)])


def _k_rowgather(table, idx, shift=0):
    f = pl.kernel(
        functools.partial(_rowgather_body, table.shape[0], shift), mesh=_mesh(),
        compiler_params=pltpu.CompilerParams(needs_layout_passes=False),
        out_type=jax.ShapeDtypeStruct((EPAD, 128), jnp.float32),
        scratch_types=[
            pltpu.VMEM((CG,), jnp.int32),
            pltpu.VMEM((CG,), jnp.int32),
            pltpu.VMEM((CG, 128), jnp.float32),
            pltpu.SemaphoreType.DMA,
        ],
    )
    return f(table, idx)


def _msg_body(xg_ref, eb_ref, a1_ref, b1a_ref, w1b_ref, b1b_ref, msg_ref):
    h = jnp.dot(xg_ref[...], a1_ref[...], preferred_element_type=jnp.float32)
    h = jnp.maximum(h + eb_ref[...] + b1a_ref[...], 0.0)
    m = jnp.dot(h, w1b_ref[...], preferred_element_type=jnp.float32) + b1b_ref[...]
    msg_ref[...] = jnp.maximum(m, 0.0)


def _k_msg(xg, eb, a1, b1a_row, w1b, b1b_row):
    return pl.pallas_call(
        _msg_body,
        grid=(EPAD // B_MSG,),
        in_specs=[
            pl.BlockSpec((B_MSG, 128), lambda i: (i, 0)),
            pl.BlockSpec((B_MSG, 64), lambda i: (i, 0)),
            pl.BlockSpec((128, 64), lambda i: (0, 0)),
            pl.BlockSpec((1, 64), lambda i: (0, 0)),
            pl.BlockSpec((64, 64), lambda i: (0, 0)),
            pl.BlockSpec((1, 64), lambda i: (0, 0)),
        ],
        out_specs=pl.BlockSpec((B_MSG, 64), lambda i: (i, 0)),
        out_shape=jax.ShapeDtypeStruct((EPAD, 64), jnp.float32),
    )(xg, eb, a1, b1a_row, w1b, b1b_row)


def _max_body(msg_hbm, pack_hbm, off_hbm, cnt_hbm, aggr_hbm,
              aggr_v, msgb0, dlb0, msgb1, dlb1, offb, cntb, sem0, sem1):
    w = _wid()
    pltpu.sync_copy(off_hbm, offb)
    pltpu.sync_copy(cnt_hbm, cntb)
    my_off = pl.multiple_of(_sget(offb, w), 16)
    my_cnt = _sget(cntb, w)

    def _zero(i, _):
        for k in range(4):
            aggr_v[pl.ds((i * 4 + k) * 16, 16)] = jnp.zeros((16,), jnp.float32)
        return 0

    lax.fori_loop(0, NPW_PAD * 64 // 64, _zero, 0)

    iota = lax.iota(jnp.int32, 16)
    nchunks = (my_cnt + CM - 1) // CM
    bufs = ((msgb0, dlb0, sem0), (msgb1, dlb1, sem1))

    def _start(c, k):
        mb, db, sm = bufs[k]
        st = pl.multiple_of(my_off + c * CM, 16)
        pltpu.async_copy(msg_hbm.at[pl.ds(st, CM)], mb, sm)
        pltpu.async_copy(pack_hbm.at[pl.ds(st, CM)], db, sm)

    def _wait(k):
        mb, db, sm = bufs[k]
        pltpu.make_async_copy(msg_hbm.at[pl.ds(0, CM)], mb, sm).wait()
        pltpu.make_async_copy(pack_hbm.at[pl.ds(0, CM)], db, sm).wait()

    def _compute(c, k):
        mb, db, _ = bufs[k]
        nvalid = my_cnt - c * CM
        for g in range(CM // 16):
            pk = db[pl.ds(g * 16, 16)]
            d64 = (pk & jnp.int32(NPW_PAD - 1)) * jnp.int32(64)
            valid = iota < (nvalid - g * 16)
            rows = iota + jnp.int32(g * 16)

            def _feat(j8, _):
                for jj in range(8):
                    f = (j8 * 8 + jj + iota) & jnp.int32(63)
                    mv = plsc.load_gather(mb, [rows, f])
                    idx = d64 + f
                    cur = plsc.load_gather(aggr_v, [idx])
                    plsc.store_scatter(aggr_v, [idx],
                                       jnp.maximum(cur, mv), mask=valid)
                return 0

            lax.fori_loop(0, 8, _feat, 0)

    @pl.when(nchunks > 0)
    def _():
        _start(0, 0)

    def _pair(c2, _):
        c0 = c2 * 2

        @pl.when(c0 < nchunks)
        def _():
            _wait(0)

            @pl.when(c0 + 1 < nchunks)
            def _():
                _start(c0 + 1, 1)

            _compute(c0, 0)

        @pl.when(c0 + 1 < nchunks)
        def _():
            _wait(1)

            @pl.when(c0 + 2 < nchunks)
            def _():
                _start(c0 + 2, 0)

            _compute(c0 + 1, 1)

        return 0

    lax.fori_loop(0, (nchunks + 1) // 2, _pair, 0)
    pltpu.sync_copy(aggr_v.at[pl.ds(0, NPW * 64)],
                    aggr_hbm.at[pl.ds(w * (NPW * 64), NPW * 64)])


def _k_max(msg, packb, off48, cnt48):
    f = pl.kernel(
        _max_body, mesh=_mesh(),
        compiler_params=pltpu.CompilerParams(needs_layout_passes=False),
        out_type=jax.ShapeDtypeStruct((NW * NPW * 64,), jnp.float32),
        scratch_types=[
            pltpu.VMEM((NPW_PAD * 64,), jnp.float32),
            pltpu.VMEM((CM, 64), jnp.float32),
            pltpu.VMEM((CM,), jnp.int32),
            pltpu.VMEM((CM, 64), jnp.float32),
            pltpu.VMEM((CM,), jnp.int32),
            pltpu.VMEM((48,), jnp.int32),
            pltpu.VMEM((48,), jnp.int32),
            pltpu.SemaphoreType.DMA,
            pltpu.SemaphoreType.DMA,
        ],
    )
    return f(msg, packb, off48, cnt48)


def _node_body(x_ref, aggr_ref, b1_ref, b2_ref, b2a_ref, w2b_ref, b2b_ref,
               g_ref, wsel_ref, out_ref):
    x = x_ref[...]
    h2 = jnp.dot(x, b1_ref[...], preferred_element_type=jnp.float32)
    h2 += jnp.dot(aggr_ref[...], b2_ref[...], preferred_element_type=jnp.float32)
    h2 = jnp.maximum(h2 + b2a_ref[...], 0.0)
    ca = jnp.dot(h2, w2b_ref[...], preferred_element_type=jnp.float32) + b2b_ref[...]
    links = ca[:, 0:1]
    addd = ca[:, 1:2]
    comb = ca[:, 2:66]
    sqrow = jnp.sum(comb * comb, axis=1, keepdims=True)
    g = g_ref[...]
    per_grp = lax.dot_general(g, sqrow, (((0,), (0,)), ((), ())),
                              preferred_element_type=jnp.float32)
    sq_g = jnp.dot(g, per_grp, preferred_element_type=jnp.float32)
    scale = lax.rsqrt(sq_g + 1e-06)
    combn = comb * scale
    out = jnp.concatenate([links, addd, combn, x[:, 0:62]], axis=1)
    wsel = wsel_ref[0]
    out = jnp.where(wsel > 0.0, out, x)
    out_ref[...] = out


def _k_node(x, aggr, b1, b2, b2a_row, w2b, b2b_row, g, wsel):
    return pl.pallas_call(
        _node_body,
        grid=(N // B_N,),
        in_specs=[
            pl.BlockSpec((B_N, 128), lambda i: (i, 0)),
            pl.BlockSpec((B_N, 64), lambda i: (i, 0)),
            pl.BlockSpec((128, 32), lambda i: (0, 0)),
            pl.BlockSpec((64, 32), lambda i: (0, 0)),
            pl.BlockSpec((1, 32), lambda i: (0, 0)),
            pl.BlockSpec((32, 66), lambda i: (0, 0)),
            pl.BlockSpec((1, 66), lambda i: (0, 0)),
            pl.BlockSpec((B_N, B_N // USERS), lambda i: (0, 0)),
            pl.BlockSpec(memory_space=pltpu.SMEM),
        ],
        out_specs=pl.BlockSpec((B_N, 128), lambda i: (i, 0)),
        out_shape=jax.ShapeDtypeStruct((N, 128), jnp.float32),
    )(x, aggr, b1, b2, b2a_row, w2b, b2b_row, g, wsel)


def kernel(x, edge_index, edge_attr, weights, W1a, b1a, W1b, b1b,
           W2a, b2a, W2b, b2b):
    src = edge_index[0]
    dst = edge_index[1]
    dst3 = dst.reshape(E // B_POS, B_POS, 1)
    a1 = W1a[:128]
    a2 = W1a[128:]
    b1 = W2a[:128]
    b2 = W2a[128:]
    b1a_row = b1a.reshape(1, 64)
    b1b_row = b1b.reshape(1, 64)
    b2a_row = b2a.reshape(1, 32)
    b2b_row = b2b.reshape(1, 66)
    lt = jnp.tril(jnp.ones((B_POS, B_POS), jnp.float32), k=-1)
    lt32 = jnp.triu(jnp.ones((NW, NW), jnp.float32), k=1)
    grp = jnp.repeat(jnp.eye(B_N // USERS, dtype=jnp.float32), USERS, axis=0)

    src3 = src.reshape(E // B_POS, B_POS, 1)
    hist = _k_hist(dst3)
    pos3, pack23, off48, cnt48 = _k_pos(dst3, src3, hist, lt, lt32)
    pos = pos3.reshape(E)
    pack2 = pack23.reshape(E)
    off48 = off48.reshape(48)
    cnt48 = cnt48.reshape(48)
    ea128 = _k_pad(edge_attr)
    packb, invp = _k_perm(pack2, pos)
    ea_b = _k_rowgather(ea128, invp)
    e_b = _k_e(ea_b, a2)

    xl = x
    for layer in range(2):
        xg = _k_rowgather(xl, packb, shift=9)
        msg = _k_msg(xg, e_b, a1, b1a_row, W1b, b1b_row)
        aggr_flat = _k_max(msg, packb, off48, cnt48)
        aggr = aggr_flat.reshape(NW * NPW, 64)[:N]
        wsel = weights[layer:layer + 1]
        xl = _k_node(xl, aggr, b1, b2, b2a_row, W2b, b2b_row, grp, wsel)
    return xl


# speedup vs baseline: 1.6647x; 1.1213x over previous
import functools

import jax
import jax.numpy as jnp
from jax import lax
from jax.experimental import pallas as pl
from jax.experimental.pallas import tpu as pltpu
from jax.experimental.pallas import tpu_sc as plsc

N = 10000
E = 320000
NW = 32
NPW = 313
NPW_PAD = 512
MULC = 107203
SHIFT = 25
EPAD = 327680
B_POS = 512
B_E = 2000
B_MSG = 2048
B_N = 2000
CP = 2000
CG = 320
CM = 256
USERS = 10
INT_MIN = -2147483647

def _mesh():
    return plsc.VectorSubcoreMesh(core_axis_name="c", subcore_axis_name="s")


def _wid():
    return lax.axis_index("s") * 2 + lax.axis_index("c")


def _sget(ref, i):
    base = (i // 16) * 16
    win = ref[pl.ds(base, 16)]
    m = lax.iota(jnp.int32, 16) == (i - base)
    return jnp.max(jnp.where(m, win, INT_MIN))


def _hist_body(dst_ref, hist_ref, acc):
    i = pl.program_id(0)

    @pl.when(i == 0)
    def _():
        acc[...] = jnp.zeros_like(acc)

    d = dst_ref[0]
    bucket = (d * MULC) >> SHIFT
    oh = jnp.where(bucket == lax.broadcasted_iota(jnp.int32, (1, NW), 1),
                   1.0, 0.0)
    acc[...] += jnp.sum(oh, axis=0, keepdims=True)

    @pl.when(i == pl.num_programs(0) - 1)
    def _():
        hist_ref[...] = acc[...]


def _k_hist(dst3):
    return pl.pallas_call(
        _hist_body,
        grid=(E // B_POS,),
        in_specs=[pl.BlockSpec((1, B_POS, 1), lambda i: (i, 0, 0))],
        out_specs=pl.BlockSpec((1, NW), lambda i: (0, 0)),
        out_shape=jax.ShapeDtypeStruct((1, NW), jnp.float32),
        scratch_shapes=[pltpu.VMEM((1, NW), jnp.float32)],
    )(dst3)


def _pos_body(dst_ref, src_ref, hist_ref, lt_ref, lt32_ref,
              pos_ref, pack_ref, off_ref, cnt_ref, run):
    i = pl.program_id(0)
    hist_i = hist_ref[...].astype(jnp.int32)
    cnt_al = (hist_i + 15) & jnp.int32(-16)
    cnt_al_f = cnt_al.astype(jnp.float32)
    off_ex_f = jnp.dot(cnt_al_f, lt32_ref[...],
                       preferred_element_type=jnp.float32)

    @pl.when(i == 0)
    def _():
        run[...] = off_ex_f
        z = jnp.zeros((1, 48 - NW - 1), jnp.float32)
        total = jnp.sum(cnt_al_f, axis=1, keepdims=True)
        off_ref[...] = jnp.concatenate([off_ex_f, total, z], axis=1).astype(jnp.int32)
        zc = jnp.zeros((1, 48 - NW), jnp.float32)
        cnt_ref[...] = jnp.concatenate([hist_ref[...], zc], axis=1).astype(jnp.int32)

    d = dst_ref[0]
    bucket = (d * MULC) >> SHIFT
    oh = jnp.where(bucket == lax.broadcasted_iota(jnp.int32, (1, NW), 1),
                   1.0, 0.0)
    rank = jnp.dot(lt_ref[...], oh, preferred_element_type=jnp.float32)
    posv = jnp.sum((rank + run[...]) * oh, axis=1, keepdims=True)
    pos_ref[0] = posv.astype(jnp.int32)
    pack_ref[0] = src_ref[0] * jnp.int32(512) + (d - bucket * jnp.int32(NPW))
    run[...] += jnp.sum(oh, axis=0, keepdims=True)


def _k_pos(dst3, src3, hist, lt, lt32):
    nb = E // B_POS
    return pl.pallas_call(
        _pos_body,
        grid=(nb,),
        in_specs=[
            pl.BlockSpec((1, B_POS, 1), lambda i: (i, 0, 0)),
            pl.BlockSpec((1, B_POS, 1), lambda i: (i, 0, 0)),
            pl.BlockSpec((1, NW), lambda i: (0, 0)),
            pl.BlockSpec((B_POS, B_POS), lambda i: (0, 0)),
            pl.BlockSpec((NW, NW), lambda i: (0, 0)),
        ],
        out_specs=[
            pl.BlockSpec((1, B_POS, 1), lambda i: (i, 0, 0)),
            pl.BlockSpec((1, B_POS, 1), lambda i: (i, 0, 0)),
            pl.BlockSpec((1, 48), lambda i: (0, 0)),
            pl.BlockSpec((1, 48), lambda i: (0, 0)),
        ],
        out_shape=[
            jax.ShapeDtypeStruct((nb, B_POS, 1), jnp.int32),
            jax.ShapeDtypeStruct((nb, B_POS, 1), jnp.int32),
            jax.ShapeDtypeStruct((1, 48), jnp.int32),
            jax.ShapeDtypeStruct((1, 48), jnp.int32),
        ],
        scratch_shapes=[pltpu.VMEM((1, NW), jnp.float32)],
    )(dst3, src3, hist, lt, lt32)


def _pad_body(ea_ref, out_ref):
    z = jnp.zeros((B_E, 128 - 65), jnp.float32)
    out_ref[...] = jnp.concatenate([ea_ref[...], z], axis=1)


def _k_pad(ea):
    return pl.pallas_call(
        _pad_body,
        grid=(E // B_E,),
        in_specs=[pl.BlockSpec((B_E, 65), lambda i: (i, 0))],
        out_specs=pl.BlockSpec((B_E, 128), lambda i: (i, 0)),
        out_shape=jax.ShapeDtypeStruct((E, 128), jnp.float32),
    )(ea)


def _e_body(ea_ref, a2_ref, e_ref):
    e_ref[...] = jnp.dot(ea_ref[:, 0:65], a2_ref[...],
                         preferred_element_type=jnp.float32)


def _k_e(ea_b, a2):
    return pl.pallas_call(
        _e_body,
        grid=(EPAD // B_MSG,),
        in_specs=[
            pl.BlockSpec((B_MSG, 128), lambda i: (i, 0)),
            pl.BlockSpec((65, 64), lambda i: (0, 0)),
        ],
        out_specs=pl.BlockSpec((B_MSG, 64), lambda i: (i, 0)),
        out_shape=jax.ShapeDtypeStruct((EPAD, 64), jnp.float32),
    )(ea_b, a2)


def _perm_body(pack_hbm, pos_hbm, packb_hbm, invp_hbm,
               packb_s, invp_s, posb, pbuf, ibuf, sem):
    s = lax.axis_index("s")
    cidx = lax.axis_index("c")
    iota = lax.iota(jnp.int32, 16)
    base = s * (E // 16)
    for c in range(E // 16 // CP):
        st = base + c * CP
        pltpu.sync_copy(pos_hbm.at[pl.ds(st, CP)], posb)
        pltpu.sync_copy(pack_hbm.at[pl.ds(st, CP)], pbuf)
        for g in range(CP // 16):
            ibuf[pl.ds(g * 16, 16)] = iota + jnp.int32(st + g * 16)
        h1 = pltpu.async_copy(pbuf, packb_s.at[posb], sem)
        h2 = pltpu.async_copy(ibuf, invp_s.at[posb], sem)
        h1.wait()
        h2.wait()
    plsc.subcore_barrier()

    @pl.when(cidx == 0)
    def _():
        sl = s * (EPAD // 16)
        pltpu.sync_copy(packb_s.at[pl.ds(sl, EPAD // 16)],
                        packb_hbm.at[pl.ds(sl, EPAD // 16)])
        pltpu.sync_copy(invp_s.at[pl.ds(sl, EPAD // 16)],
                        invp_hbm.at[pl.ds(sl, EPAD // 16)])


def _k_perm(pack2, pos):
    f = pl.kernel(
        _perm_body, mesh=_mesh(),
        compiler_params=pltpu.CompilerParams(needs_layout_passes=False),
        out_type=(
            jax.ShapeDtypeStruct((EPAD,), jnp.int32),
            jax.ShapeDtypeStruct((EPAD,), jnp.int32),
        ),
        scratch_types=[
            pltpu.VMEM_SHARED((EPAD,), jnp.int32),
            pltpu.VMEM_SHARED((EPAD,), jnp.int32),
            pltpu.VMEM((CP,), jnp.int32),
            pltpu.VMEM((CP,), jnp.int32),
            pltpu.VMEM((CP,), jnp.int32),
            pltpu.SemaphoreType.DMA,
        ],
    )
    return f(pack2, pos)


def _rowgather_body(nrows, shift, stage, table_hbm, idx_hbm, out_hbm,
                    table_s, idxb, idxc, rows, gsem, wsem):
    w = _wid()
    if stage:
        @pl.when(lax.axis_index("s") == 0)
        def _():
            pltpu.sync_copy(table_hbm, table_s)
        plsc.subcore_barrier()
        table = table_s
    else:
        table = table_hbm
    base = w * (EPAD // NW)
    nch = EPAD // NW // CG

    def _load_idx(c, k):
        st = base + c * CG
        pltpu.sync_copy(idx_hbm.at[pl.ds(st, CG)], idxb[k])
        for g in range(CG // 16):
            v = idxb[k][pl.ds(g * 16, 16)] >> shift
            v = jnp.minimum(jnp.maximum(v, 0), jnp.int32(nrows - 1))
            idxc[k][pl.ds(g * 16, 16)] = v

    _load_idx(0, 0)
    pltpu.async_copy(table.at[idxc[0]], rows[0], gsem[0])
    for c in range(nch):
        k = c & 1
        if c + 1 < nch:
            _load_idx(c + 1, k ^ 1)
            if c >= 1:
                pltpu.make_async_copy(
                    rows[k ^ 1], out_hbm.at[pl.ds(base + (c - 1) * CG, CG)],
                    wsem[k ^ 1]).wait()
            pltpu.async_copy(table.at[idxc[k ^ 1]], rows[k ^ 1], gsem[k ^ 1])
        pltpu.make_async_copy(table.at[idxc[k]], rows[k], gsem[k]).wait()
        pltpu.async_copy(rows[k], out_hbm.at[pl.ds(base + c * CG, CG)], wsem[k])
    pltpu.make_async_copy(
        rows[(nch - 1) & 1],
        out_hbm.at[pl.ds(base + (nch - 1) * CG, CG)], wsem[(nch - 1) & 1]).wait()
    if nch >= 2:
        pltpu.make_async_copy(
            rows[(nch - 2) & 1],
            out_hbm.at[pl.ds(base + (nch - 2) * CG, CG)], wsem[(nch - 2) & 1]).wait()


def _k_rowgather(table, idx, shift=0, stage=False):
    nrows = table.shape[0]
    f = pl.kernel(
        functools.partial(_rowgather_body, nrows, shift, stage), mesh=_mesh(),
        compiler_params=pltpu.CompilerParams(needs_layout_passes=False),
        out_type=jax.ShapeDtypeStruct((EPAD, 128), jnp.float32),
        scratch_types=[
            (pltpu.VMEM_SHARED((nrows, 128), jnp.float32) if stage else None),
            [pltpu.VMEM((CG,), jnp.int32)] * 2,
            [pltpu.VMEM((CG,), jnp.int32)] * 2,
            [pltpu.VMEM((CG, 128), jnp.float32)] * 2,
            [pltpu.SemaphoreType.DMA] * 2,
            [pltpu.SemaphoreType.DMA] * 2,
        ],
    )
    return f(table, idx)


def _msg_body(xg_ref, eb_ref, a1_ref, b1a_ref, w1b_ref, b1b_ref, msg_ref):
    h = jnp.dot(xg_ref[...], a1_ref[...], preferred_element_type=jnp.float32)
    h = jnp.maximum(h + eb_ref[...] + b1a_ref[...], 0.0)
    m = jnp.dot(h, w1b_ref[...], preferred_element_type=jnp.float32) + b1b_ref[...]
    msg_ref[...] = jnp.maximum(m, 0.0)


def _k_msg(xg, eb, a1, b1a_row, w1b, b1b_row):
    return pl.pallas_call(
        _msg_body,
        grid=(EPAD // B_MSG,),
        in_specs=[
            pl.BlockSpec((B_MSG, 128), lambda i: (i, 0)),
            pl.BlockSpec((B_MSG, 64), lambda i: (i, 0)),
            pl.BlockSpec((128, 64), lambda i: (0, 0)),
            pl.BlockSpec((1, 64), lambda i: (0, 0)),
            pl.BlockSpec((64, 64), lambda i: (0, 0)),
            pl.BlockSpec((1, 64), lambda i: (0, 0)),
        ],
        out_specs=pl.BlockSpec((B_MSG, 64), lambda i: (i, 0)),
        out_shape=jax.ShapeDtypeStruct((EPAD, 64), jnp.float32),
    )(xg, eb, a1, b1a_row, w1b, b1b_row)


def _max_body(msg_hbm, pack_hbm, off_hbm, cnt_hbm, aggr_hbm,
              aggr_v, msgb0, dlb0, msgb1, dlb1, offb, cntb, sem0, sem1):
    w = _wid()
    pltpu.sync_copy(off_hbm, offb)
    pltpu.sync_copy(cnt_hbm, cntb)
    my_off = pl.multiple_of(_sget(offb, w), 16)
    my_cnt = _sget(cntb, w)

    def _zero(i, _):
        for k in range(4):
            aggr_v[pl.ds((i * 4 + k) * 16, 16)] = jnp.zeros((16,), jnp.float32)
        return 0

    lax.fori_loop(0, NPW_PAD * 64 // 64, _zero, 0)

    iota = lax.iota(jnp.int32, 16)
    nchunks = (my_cnt + CM - 1) // CM
    bufs = ((msgb0, dlb0, sem0), (msgb1, dlb1, sem1))

    def _start(c, k):
        mb, db, sm = bufs[k]
        st = pl.multiple_of(my_off + c * CM, 16)
        pltpu.async_copy(msg_hbm.at[pl.ds(st, CM)], mb, sm)
        pltpu.async_copy(pack_hbm.at[pl.ds(st, CM)], db, sm)

    def _wait(k):
        mb, db, sm = bufs[k]
        pltpu.make_async_copy(msg_hbm.at[pl.ds(0, CM)], mb, sm).wait()
        pltpu.make_async_copy(pack_hbm.at[pl.ds(0, CM)], db, sm).wait()

    def _compute(c, k):
        mb, db, _ = bufs[k]
        nvalid = my_cnt - c * CM
        for g in range(CM // 16):
            pk = db[pl.ds(g * 16, 16)]
            d64 = (pk & jnp.int32(NPW_PAD - 1)) * jnp.int32(64)
            valid = iota < (nvalid - g * 16)
            rows = iota + jnp.int32(g * 16)

            def _feat(j8, _):
                for jj in range(8):
                    f = (j8 * 8 + jj + iota) & jnp.int32(63)
                    mv = plsc.load_gather(mb, [rows, f])
                    idx = d64 + f
                    cur = plsc.load_gather(aggr_v, [idx])
                    plsc.store_scatter(aggr_v, [idx],
                                       jnp.maximum(cur, mv), mask=valid)
                return 0

            lax.fori_loop(0, 8, _feat, 0)

    @pl.when(nchunks > 0)
    def _():
        _start(0, 0)

    def _pair(c2, _):
        c0 = c2 * 2

        @pl.when(c0 < nchunks)
        def _():
            _wait(0)

            @pl.when(c0 + 1 < nchunks)
            def _():
                _start(c0 + 1, 1)

            _compute(c0, 0)

        @pl.when(c0 + 1 < nchunks)
        def _():
            _wait(1)

            @pl.when(c0 + 2 < nchunks)
            def _():
                _start(c0 + 2, 0)

            _compute(c0 + 1, 1)

        return 0

    lax.fori_loop(0, (nchunks + 1) // 2, _pair, 0)
    pltpu.sync_copy(aggr_v.at[pl.ds(0, NPW * 64)],
                    aggr_hbm.at[pl.ds(w * (NPW * 64), NPW * 64)])


def _k_max(msg, packb, off48, cnt48):
    f = pl.kernel(
        _max_body, mesh=_mesh(),
        compiler_params=pltpu.CompilerParams(needs_layout_passes=False),
        out_type=jax.ShapeDtypeStruct((NW * NPW * 64,), jnp.float32),
        scratch_types=[
            pltpu.VMEM((NPW_PAD * 64,), jnp.float32),
            pltpu.VMEM((CM, 64), jnp.float32),
            pltpu.VMEM((CM,), jnp.int32),
            pltpu.VMEM((CM, 64), jnp.float32),
            pltpu.VMEM((CM,), jnp.int32),
            pltpu.VMEM((48,), jnp.int32),
            pltpu.VMEM((48,), jnp.int32),
            pltpu.SemaphoreType.DMA,
            pltpu.SemaphoreType.DMA,
        ],
    )
    return f(msg, packb, off48, cnt48)


def _node_body(x_ref, aggr_ref, b1_ref, b2_ref, b2a_ref, w2b_ref, b2b_ref,
               g_ref, wsel_ref, out_ref):
    x = x_ref[...]
    h2 = jnp.dot(x, b1_ref[...], preferred_element_type=jnp.float32)
    h2 += jnp.dot(aggr_ref[...], b2_ref[...], preferred_element_type=jnp.float32)
    h2 = jnp.maximum(h2 + b2a_ref[...], 0.0)
    ca = jnp.dot(h2, w2b_ref[...], preferred_element_type=jnp.float32) + b2b_ref[...]
    links = ca[:, 0:1]
    addd = ca[:, 1:2]
    comb = ca[:, 2:66]
    sqrow = jnp.sum(comb * comb, axis=1, keepdims=True)
    g = g_ref[...]
    per_grp = lax.dot_general(g, sqrow, (((0,), (0,)), ((), ())),
                              preferred_element_type=jnp.float32)
    sq_g = jnp.dot(g, per_grp, preferred_element_type=jnp.float32)
    scale = lax.rsqrt(sq_g + 1e-06)
    combn = comb * scale
    out = jnp.concatenate([links, addd, combn, x[:, 0:62]], axis=1)
    wsel = wsel_ref[0]
    out = jnp.where(wsel > 0.0, out, x)
    out_ref[...] = out


def _k_node(x, aggr, b1, b2, b2a_row, w2b, b2b_row, g, wsel):
    return pl.pallas_call(
        _node_body,
        grid=(N // B_N,),
        in_specs=[
            pl.BlockSpec((B_N, 128), lambda i: (i, 0)),
            pl.BlockSpec((B_N, 64), lambda i: (i, 0)),
            pl.BlockSpec((128, 32), lambda i: (0, 0)),
            pl.BlockSpec((64, 32), lambda i: (0, 0)),
            pl.BlockSpec((1, 32), lambda i: (0, 0)),
            pl.BlockSpec((32, 66), lambda i: (0, 0)),
            pl.BlockSpec((1, 66), lambda i: (0, 0)),
            pl.BlockSpec((B_N, B_N // USERS), lambda i: (0, 0)),
            pl.BlockSpec(memory_space=pltpu.SMEM),
        ],
        out_specs=pl.BlockSpec((B_N, 128), lambda i: (i, 0)),
        out_shape=jax.ShapeDtypeStruct((N, 128), jnp.float32),
    )(x, aggr, b1, b2, b2a_row, w2b, b2b_row, g, wsel)


def kernel(x, edge_index, edge_attr, weights, W1a, b1a, W1b, b1b,
           W2a, b2a, W2b, b2b):
    src = edge_index[0]
    dst = edge_index[1]
    dst3 = dst.reshape(E // B_POS, B_POS, 1)
    a1 = W1a[:128]
    a2 = W1a[128:]
    b1 = W2a[:128]
    b2 = W2a[128:]
    b1a_row = b1a.reshape(1, 64)
    b1b_row = b1b.reshape(1, 64)
    b2a_row = b2a.reshape(1, 32)
    b2b_row = b2b.reshape(1, 66)
    lt = jnp.tril(jnp.ones((B_POS, B_POS), jnp.float32), k=-1)
    lt32 = jnp.triu(jnp.ones((NW, NW), jnp.float32), k=1)
    grp = jnp.repeat(jnp.eye(B_N // USERS, dtype=jnp.float32), USERS, axis=0)

    src3 = src.reshape(E // B_POS, B_POS, 1)
    hist = _k_hist(dst3)
    pos3, pack23, off48, cnt48 = _k_pos(dst3, src3, hist, lt, lt32)
    pos = pos3.reshape(E)
    pack2 = pack23.reshape(E)
    off48 = off48.reshape(48)
    cnt48 = cnt48.reshape(48)
    ea128 = _k_pad(edge_attr)
    packb, invp = _k_perm(pack2, pos)
    ea_b = _k_rowgather(ea128, invp)
    e_b = _k_e(ea_b, a2)

    xl = x
    for layer in range(2):
        xg = _k_rowgather(xl, packb, shift=9)
        msg = _k_msg(xg, e_b, a1, b1a_row, W1b, b1b_row)
        aggr_flat = _k_max(msg, packb, off48, cnt48)
        aggr = aggr_flat.reshape(NW * NPW, 64)[:N]
        wsel = weights[layer:layer + 1]
        xl = _k_node(xl, aggr, b1, b2, b2a_row, W2b, b2b_row, grp, wsel)
    return xl
